# baseline (speedup 1.0000x reference)
import jax
import jax.numpy as jnp
from jax.experimental import pallas as pl
from jax.experimental.pallas import tpu as pltpu


def _mlp_block(x_ref, m_ref, W1r, b1r, W2r, b2r, W3sr, bfr, Wc2r, bc2r,
               out_ref):
    nh = x_ref.shape[1]
    h2s = []
    for p in range(nh // 2):
        xp = jnp.concatenate([x_ref[0, 2 * p], x_ref[0, 2 * p + 1]],
                             axis=-1).astype(jnp.bfloat16)
        parts = [jnp.dot(xp, W1r[p, j], preferred_element_type=jnp.float32)
                 for j in range(4)]
        h1 = jnp.concatenate(parts, axis=0)
        h1 = jnp.maximum(h1 + b1r[p], 0.0).astype(jnp.bfloat16)
        h2 = jnp.dot(h1, W2r[p], preferred_element_type=jnp.float32)
        h2 = jnp.maximum(h2 + b2r[p], 0.0).astype(jnp.bfloat16)
        h2s.append(h2)
    h2all = jnp.concatenate(h2s, axis=-1)
    acc = jnp.dot(h2all, W3sr[...], preferred_element_type=jnp.float32)
    hc = jnp.maximum(acc + bfr[0], 0.0).astype(jnp.bfloat16)
    oc = jnp.dot(hc, Wc2r[...], preferred_element_type=jnp.float32) + bc2r[0]
    r4 = x_ref.shape[2]
    out_ref[0] = oc.reshape(4, r4, oc.shape[-1]) * m_ref[0]


def kernel(stacks, mask, W1, b1, W2, b2, W3, b3, Wc1, bc1, Wc2, bc2):
    B, NH, N, _, D = stacks.shape
    H = W1.shape[-1]
    HC = Wc1.shape[-1]
    DOUT = Wc2.shape[-1]
    NN = N * N
    NP = NH // 2
    NN4 = NN // 4

    R = NN
    for cand in (2048, 1024, 512, 256, 128, 64, 32, 16, 8):
        if NN % cand == 0:
            R = cand
            break
    R4 = R // 4

    xs4 = stacks.reshape(B, NH, NN4, 4 * D)
    mfp = (mask.astype(jnp.float32).reshape(B, NN4, 4)
           .transpose(0, 2, 1).reshape(B, 4, NN4, 1))

    W1sh = jnp.zeros((NH, 4, 4 * D, H), jnp.float32)
    for j in range(4):
        W1sh = W1sh.at[:, j, j * D:(j + 1) * D, :].set(W1)
    zA = jnp.zeros((NP, 4, 4 * D, H), jnp.float32)
    W1p = jnp.concatenate([
        jnp.concatenate([W1sh[0::2], zA], axis=-1),
        jnp.concatenate([zA, W1sh[1::2]], axis=-1),
    ], axis=-2).astype(jnp.bfloat16)
    b1p = b1.reshape(NP, 2 * H)
    zB = jnp.zeros((NP, H, H), jnp.float32)
    W2bd = jnp.concatenate([
        jnp.concatenate([W2[0::2], zB], axis=2),
        jnp.concatenate([zB, W2[1::2]], axis=2),
    ], axis=1).astype(jnp.bfloat16)
    b2p = b2.reshape(NP, 2 * H)
    W3s = (W3 * Wc1[:, None, :]).reshape(NH * H, HC).astype(jnp.bfloat16)
    bf = (bc1 + jnp.sum(b3 * Wc1, axis=0)).reshape(1, HC)
    Wc2b = Wc2.astype(jnp.bfloat16)
    bc2r = bc2.reshape(1, DOUT)

    grid = (B, NN4 // R4)
    full = lambda shape: pl.BlockSpec(shape, lambda b, j: (0,) * len(shape))
    out = pl.pallas_call(
        _mlp_block,
        grid=grid,
        in_specs=[
            pl.BlockSpec((1, NH, R4, 4 * D), lambda b, j: (b, 0, j, 0)),
            pl.BlockSpec((1, 4, R4, 1), lambda b, j: (b, 0, j, 0)),
            full(W1p.shape), full(b1p.shape), full(W2bd.shape),
            full(b2p.shape), full(W3s.shape), full(bf.shape),
            full(Wc2b.shape), full(bc2r.shape),
        ],
        out_specs=pl.BlockSpec((1, 4, R4, DOUT), lambda b, j: (b, 0, j, 0)),
        out_shape=jax.ShapeDtypeStruct((B, 4, NN4, DOUT), jnp.float32),
        compiler_params=pltpu.CompilerParams(
            dimension_semantics=("parallel", "parallel")),
    )(xs4, mfp, W1p, b1p, W2bd, b2p, W3s, bf, Wc2b, bc2r)
    return out.transpose(0, 2, 1, 3).reshape(B, N, N, DOUT)


# speedup vs baseline: 1.5306x; 1.5306x over previous
import jax
import jax.numpy as jnp
from jax.experimental import pallas as pl
from jax.experimental.pallas import tpu as pltpu


def _mlp_block(x_ref, m_ref, W1r, b1r, W2r, b2r, W3sr, bfr, Wc2r, bc2r,
               out_ref):
    nh = x_ref.shape[1]
    nr, n, d = x_ref.shape[2], x_ref.shape[3], x_ref.shape[4]
    r = nr * n
    h2s = []
    for p in range(nh // 2):
        h1s = []
        for q in (2 * p, 2 * p + 1):
            x = x_ref[0, q].reshape(r, d).astype(jnp.bfloat16)
            h1 = jnp.dot(x, W1r[q], preferred_element_type=jnp.float32)
            h1 = jnp.maximum(h1 + b1r[q], 0.0).astype(jnp.bfloat16)
            h1s.append(h1)
        h1pair = jnp.concatenate(h1s, axis=-1)
        h2 = jnp.dot(h1pair, W2r[p], preferred_element_type=jnp.float32)
        h2 = jnp.maximum(h2 + b2r[p], 0.0).astype(jnp.bfloat16)
        h2s.append(h2)
    h2all = jnp.concatenate(h2s, axis=-1)
    acc = jnp.dot(h2all, W3sr[...], preferred_element_type=jnp.float32)
    hc = jnp.maximum(acc + bfr[0], 0.0).astype(jnp.bfloat16)
    oc = jnp.dot(hc, Wc2r[...], preferred_element_type=jnp.float32) + bc2r[0]
    out_ref[0] = oc.reshape(nr, n, oc.shape[-1]) * m_ref[0]


def kernel(stacks, mask, W1, b1, W2, b2, W3, b3, Wc1, bc1, Wc2, bc2):
    B, NH, N, _, D = stacks.shape
    H = W1.shape[-1]
    HC = Wc1.shape[-1]
    DOUT = Wc2.shape[-1]
    NP = NH // 2

    NR = max(1, min(N, 2048 // N))
    while N % NR:
        NR -= 1

    mf = mask[..., None].astype(jnp.float32)

    W1b = W1.astype(jnp.bfloat16)
    z = jnp.zeros((NP, H, H), jnp.float32)
    W2bd = jnp.concatenate([
        jnp.concatenate([W2[0::2], z], axis=2),
        jnp.concatenate([z, W2[1::2]], axis=2),
    ], axis=1).astype(jnp.bfloat16)
    b2p = b2.reshape(NP, 2 * H)
    W3s = (W3 * Wc1[:, None, :]).reshape(NH * H, HC).astype(jnp.bfloat16)
    bf = (bc1 + jnp.sum(b3 * Wc1, axis=0)).reshape(1, HC)
    Wc2b = Wc2.astype(jnp.bfloat16)
    bc2r = bc2.reshape(1, DOUT)

    grid = (B, N // NR)
    full = lambda shape: pl.BlockSpec(shape, lambda b, j: (0,) * len(shape))
    out = pl.pallas_call(
        _mlp_block,
        grid=grid,
        in_specs=[
            pl.BlockSpec((1, NH, NR, N, D), lambda b, j: (b, 0, j, 0, 0)),
            pl.BlockSpec((1, NR, N, 1), lambda b, j: (b, j, 0, 0)),
            full(W1b.shape), full(b1.shape), full(W2bd.shape), full(b2p.shape),
            full(W3s.shape), full(bf.shape), full(Wc2b.shape),
            full(bc2r.shape),
        ],
        out_specs=pl.BlockSpec((1, NR, N, DOUT), lambda b, j: (b, j, 0, 0)),
        out_shape=jax.ShapeDtypeStruct((B, N, N, DOUT), jnp.float32),
        compiler_params=pltpu.CompilerParams(
            dimension_semantics=("parallel", "parallel")),
    )(stacks, mf, W1b, b1, W2bd, b2p, W3s, bf, Wc2b, bc2r)
    return out
